# baseline (speedup 1.0000x reference)
import functools

import jax
import jax.numpy as jnp
from jax import lax
from jax.experimental import pallas as pl
from jax.experimental.pallas import tpu as pltpu
from jax.experimental.pallas import tpu_sc as plsc

N = 10000
E = 160000
D = 256
R = 7
L = 3
H = 128
NTILES = 16
ET = E // NTILES
B = 50
NB = ET // B
SB = 20
NSB = NB // SB
NIB = 3
CHUNK = 640
LAST = N - CHUNK * (NTILES - 1)


NROW = (R + 1) * N


def _yz_body(h_ref, w_ref, b_ref, y_ref):
    acc = jnp.dot(h_ref[...].astype(jnp.bfloat16),
                  w_ref[0].astype(jnp.bfloat16),
                  preferred_element_type=jnp.float32)
    sel = jnp.where(pl.program_id(1) == R, 1.0, 0.0)
    acc = acc + sel * b_ref[...]
    y_ref[0] = acc[:, :H]
    y_ref[1] = acc[:, H:]


def _tc_tables(h, w_cat, b):
    bn = 2000
    nb = N // bn
    return pl.pallas_call(
        _yz_body,
        grid=(nb, R + 1),
        in_specs=[
            pl.BlockSpec((bn, D), lambda i, r: (i, 0)),
            pl.BlockSpec((1, D, D), lambda i, r: (r, 0, 0)),
            pl.BlockSpec((1, D), lambda i, r: (0, 0)),
        ],
        out_specs=pl.BlockSpec((2, bn, H), lambda i, r: (0, r * nb + i, 0)),
        out_shape=jax.ShapeDtypeStruct((2, NROW, H), jnp.float32),
    )(h, w_cat, b.reshape(1, D))


def _combine_body(a_ref, o_ref):
    o_ref[...] = jnp.maximum(
        jnp.concatenate([a_ref[0], a_ref[1]], axis=-1), 0.0)


def _tc_relu_concat(acc2):
    bn = 2000
    nb = N // bn
    return pl.pallas_call(
        _combine_body,
        grid=(nb,),
        in_specs=[pl.BlockSpec((2, bn, H), lambda i: (0, i, 0))],
        out_specs=pl.BlockSpec((bn, D), lambda i: (i, 0)),
        out_shape=jax.ShapeDtypeStruct((N, D), jnp.float32),
    )(acc2)


NBUF = 4


def _sc_scatter_body(y_hbm, idx_hbm, out_hbm,
                     ibufs, isems, bufs, gsems, ssems, acc):
    c = lax.axis_index("c")
    s = lax.axis_index("s")
    pltpu.async_copy(idx_hbm.at[c, s, 0], ibufs[0], isems[0])
    zbase = c * NROW + R * N

    @pl.when(s < NTILES - 1)
    def _():
        pltpu.sync_copy(y_hbm.at[pl.ds(zbase + s * CHUNK, CHUNK)],
                        acc.at[pl.ds(s * CHUNK, CHUNK)])

    @pl.when(s == NTILES - 1)
    def _():
        pltpu.sync_copy(
            y_hbm.at[pl.ds(zbase + (NTILES - 1) * CHUNK, LAST)],
            acc.at[pl.ds((NTILES - 1) * CHUNK, LAST)])

    pltpu.make_async_copy(idx_hbm.at[c, s, 0], ibufs[0], isems[0]).wait()
    pltpu.async_copy(idx_hbm.at[c, s, 1], ibufs[1], isems[1])
    plsc.subcore_barrier()

    for p in range(2):
        pltpu.async_copy(y_hbm.at[ibufs[0].at[p, 0]], bufs[p], gsems[p])

    def _step(j, bl, p, ib, ib2, bl2, p2):
        pltpu.make_async_copy(y_hbm.at[ib.at[bl, 0]],
                              bufs[p], gsems[p]).wait()
        pltpu.async_copy(bufs[p], acc.at[ib.at[bl, 1]], ssems[p], add=True)

        def _drain():
            pltpu.make_async_copy(bufs[p2], acc.at[ib2.at[bl2, 1]],
                                  ssems[p2]).wait()

        if isinstance(j, int):
            if j >= 2:
                _drain()
        else:
            pl.when(j >= 2)(_drain)

        pltpu.async_copy(y_hbm.at[ib2.at[bl2, 0]], bufs[p2], gsems[p2])

    for k in range(NSB):
        ib = ibufs[k % NIB]
        ibn = ibufs[(k + 1) % NIB]

        def inner(g, carry, ib=ib, k=k):
            for p in range(NBUF):
                bl = g * NBUF + p
                _step(k * SB + bl, bl, p, ib, ib, bl + 2, (p + 2) % NBUF)
            return carry

        lax.fori_loop(0, (SB - NBUF) // NBUF, inner, 0)
        if k + 1 < NSB:
            pltpu.make_async_copy(idx_hbm.at[c, s, k + 1], ibn,
                                  isems[(k + 1) % NIB]).wait()
        for p in range(NBUF):
            bl = SB - NBUF + p
            j = k * SB + bl
            if bl + 2 < SB:
                _step(j, bl, p, ib, ib, bl + 2, (p + 2) % NBUF)
            elif k + 1 < NSB:
                _step(j, bl, p, ib, ibn, bl + 2 - SB, (p + 2) % NBUF)
            else:
                pltpu.make_async_copy(y_hbm.at[ib.at[bl, 0]],
                                      bufs[p], gsems[p]).wait()
                pltpu.async_copy(bufs[p], acc.at[ib.at[bl, 1]],
                                 ssems[p], add=True)
        if k + 2 < NSB:
            pltpu.async_copy(idx_hbm.at[c, s, k + 2],
                             ibufs[(k + 2) % NIB], isems[(k + 2) % NIB])
    for p in range(NBUF):
        bl = SB - NBUF + p
        pltpu.make_async_copy(bufs[p],
                              acc.at[ibufs[(NSB - 1) % NIB].at[bl, 1]],
                              ssems[p]).wait()
    plsc.subcore_barrier()

    @pl.when(s < NTILES - 1)
    def _():
        pltpu.sync_copy(acc.at[pl.ds(s * CHUNK, CHUNK)],
                        out_hbm.at[c, pl.ds(s * CHUNK, CHUNK)])

    @pl.when(s == NTILES - 1)
    def _():
        pltpu.sync_copy(acc.at[pl.ds((NTILES - 1) * CHUNK, LAST)],
                        out_hbm.at[c, pl.ds((NTILES - 1) * CHUNK, LAST)])


@functools.cache
def _make_sc_scatter():
    return pl.kernel(
        _sc_scatter_body,
        out_type=jax.ShapeDtypeStruct((2, N, H), jnp.float32),
        mesh=plsc.VectorSubcoreMesh(core_axis_name="c", subcore_axis_name="s",
                                    num_cores=2, num_subcores=NTILES),
        scratch_types=[
            tuple(pltpu.VMEM((SB, 2, B), jnp.int32) for _ in range(NIB)),
            tuple(pltpu.SemaphoreType.DMA for _ in range(NIB)),
            tuple(pltpu.VMEM((B, H), jnp.float32) for _ in range(NBUF)),
            tuple(pltpu.SemaphoreType.DMA for _ in range(NBUF)),
            tuple(pltpu.SemaphoreType.DMA for _ in range(NBUF)),
            pltpu.VMEM_SHARED((N, H), jnp.float32),
        ],
    )


def kernel(x, edge_index, edge_type, node_position, W_rel, W_self, b):
    src = edge_index[0]
    dst = edge_index[1]
    tidx = edge_type * N + src
    ge = jnp.stack([tidx, tidx + NROW]).reshape(2, NTILES, NSB, SB, B)
    de = jnp.broadcast_to(dst, (2, E)).reshape(2, NTILES, NSB, SB, B)
    idx = jnp.stack([ge, de], axis=-2)
    w_cat = jnp.concatenate([W_rel, W_self[:, None]], axis=1)

    h = x
    outs = []
    for l in range(L):
        yz = _tc_tables(h, w_cat[l], b[l])
        acc2 = _make_sc_scatter()(yz.reshape(2 * NROW, H), idx)
        h = _tc_relu_concat(acc2)
        outs.append(h)
    node_feature = jnp.concatenate(outs, axis=-1)
    return node_feature, node_position


# speedup vs baseline: 1.1794x; 1.1794x over previous
import functools

import jax
import jax.numpy as jnp
from jax import lax
from jax.experimental import pallas as pl
from jax.experimental.pallas import tpu as pltpu
from jax.experimental.pallas import tpu_sc as plsc

N = 10000
E = 160000
D = 256
R = 7
L = 3
H = 128
NTILES = 16
ET = E // NTILES
B = 125
NB = ET // B
SB = 10
NSB = NB // SB
CHUNK = 640
LAST = N - CHUNK * (NTILES - 1)


NROW = (R + 1) * N


def _yz_body(h_ref, w_ref, b_ref, y_ref):
    acc = jnp.dot(h_ref[...].astype(jnp.bfloat16),
                  w_ref[0].astype(jnp.bfloat16),
                  preferred_element_type=jnp.float32)
    sel = jnp.where(pl.program_id(1) == R, 1.0, 0.0)
    acc = acc + sel * b_ref[...]
    y_ref[0] = acc[:, :H]
    y_ref[1] = acc[:, H:]


def _tc_tables(h, w_cat, b):
    bn = 2000
    nb = N // bn
    return pl.pallas_call(
        _yz_body,
        grid=(nb, R + 1),
        in_specs=[
            pl.BlockSpec((bn, D), lambda i, r: (i, 0)),
            pl.BlockSpec((1, D, D), lambda i, r: (r, 0, 0)),
            pl.BlockSpec((1, D), lambda i, r: (0, 0)),
        ],
        out_specs=pl.BlockSpec((2, bn, H), lambda i, r: (0, r * nb + i, 0)),
        out_shape=jax.ShapeDtypeStruct((2, NROW, H), jnp.float32),
    )(h, w_cat, b.reshape(1, D))


def _yzc_body(a_ref, w_ref, b_ref, y_ref, h_ref):
    h = jnp.maximum(jnp.concatenate([a_ref[0], a_ref[1]], axis=-1), 0.0)
    h_ref[...] = h
    acc = jnp.dot(h.astype(jnp.bfloat16), w_ref[0].astype(jnp.bfloat16),
                  preferred_element_type=jnp.float32)
    sel = jnp.where(pl.program_id(1) == R, 1.0, 0.0)
    acc = acc + sel * b_ref[...]
    y_ref[0] = acc[:, :H]
    y_ref[1] = acc[:, H:]


def _tc_tables_fused(acc2, w_cat, b):
    bn = 2000
    nb = N // bn
    return pl.pallas_call(
        _yzc_body,
        grid=(nb, R + 1),
        in_specs=[
            pl.BlockSpec((2, bn, H), lambda i, r: (0, i, 0)),
            pl.BlockSpec((1, D, D), lambda i, r: (r, 0, 0)),
            pl.BlockSpec((1, D), lambda i, r: (0, 0)),
        ],
        out_specs=[
            pl.BlockSpec((2, bn, H), lambda i, r: (0, r * nb + i, 0)),
            pl.BlockSpec((bn, D), lambda i, r: (i, 0)),
        ],
        out_shape=[
            jax.ShapeDtypeStruct((2, NROW, H), jnp.float32),
            jax.ShapeDtypeStruct((N, D), jnp.float32),
        ],
    )(acc2, w_cat, b.reshape(1, D))


def _combine_body(a_ref, o_ref):
    o_ref[...] = jnp.maximum(
        jnp.concatenate([a_ref[0], a_ref[1]], axis=-1), 0.0)


def _tc_relu_concat(acc2):
    bn = 2000
    nb = N // bn
    return pl.pallas_call(
        _combine_body,
        grid=(nb,),
        in_specs=[pl.BlockSpec((2, bn, H), lambda i: (0, i, 0))],
        out_specs=pl.BlockSpec((bn, D), lambda i: (i, 0)),
        out_shape=jax.ShapeDtypeStruct((N, D), jnp.float32),
    )(acc2)


NBUF = 2


def _sc_scatter_body(y_hbm, idx_hbm, out_hbm,
                     ibufs, isems, bufs, sems, acc):
    c = lax.axis_index("c")
    s = lax.axis_index("s")
    pltpu.async_copy(idx_hbm.at[c, s, 0], ibufs[0], isems[0])
    zbase = c * NROW + R * N

    @pl.when(s < NTILES - 1)
    def _():
        pltpu.sync_copy(y_hbm.at[pl.ds(zbase + s * CHUNK, CHUNK)],
                        acc.at[pl.ds(s * CHUNK, CHUNK)])

    @pl.when(s == NTILES - 1)
    def _():
        pltpu.sync_copy(
            y_hbm.at[pl.ds(zbase + (NTILES - 1) * CHUNK, LAST)],
            acc.at[pl.ds((NTILES - 1) * CHUNK, LAST)])

    pltpu.make_async_copy(idx_hbm.at[c, s, 0], ibufs[0], isems[0]).wait()
    pltpu.async_copy(idx_hbm.at[c, s, 1], ibufs[1], isems[1])
    plsc.subcore_barrier()

    for p in range(NBUF):
        pltpu.async_copy(y_hbm.at[ibufs[0].at[p, 0]], bufs[p], sems[p])

    for k in range(NSB):
        ib = ibufs[k % 2]
        ibn = ibufs[(k + 1) % 2]

        def inner(b2, carry, ib=ib):
            for p in range(NBUF):
                bl = b2 * NBUF + p
                pltpu.make_async_copy(y_hbm.at[ib.at[bl, 0]],
                                      bufs[p], sems[p]).wait()
                pltpu.sync_copy(bufs[p], acc.at[ib.at[bl, 1]], add=True)
                pltpu.async_copy(y_hbm.at[ib.at[bl + NBUF, 0]],
                                 bufs[p], sems[p])
            return carry

        lax.fori_loop(0, (SB - NBUF) // NBUF, inner, 0)
        if k + 1 < NSB:
            pltpu.make_async_copy(idx_hbm.at[c, s, k + 1], ibn,
                                  isems[(k + 1) % 2]).wait()
        for p in range(NBUF):
            bl = SB - NBUF + p
            pltpu.make_async_copy(y_hbm.at[ib.at[bl, 0]],
                                  bufs[p], sems[p]).wait()
            pltpu.sync_copy(bufs[p], acc.at[ib.at[bl, 1]], add=True)
            if k + 1 < NSB:
                pltpu.async_copy(y_hbm.at[ibn.at[p, 0]], bufs[p], sems[p])
        if k + 2 < NSB:
            pltpu.async_copy(idx_hbm.at[c, s, k + 2], ib, isems[k % 2])
    plsc.subcore_barrier()

    @pl.when(s < NTILES - 1)
    def _():
        pltpu.sync_copy(acc.at[pl.ds(s * CHUNK, CHUNK)],
                        out_hbm.at[c, pl.ds(s * CHUNK, CHUNK)])

    @pl.when(s == NTILES - 1)
    def _():
        pltpu.sync_copy(acc.at[pl.ds((NTILES - 1) * CHUNK, LAST)],
                        out_hbm.at[c, pl.ds((NTILES - 1) * CHUNK, LAST)])


@functools.cache
def _make_sc_scatter():
    return pl.kernel(
        _sc_scatter_body,
        out_type=jax.ShapeDtypeStruct((2, N, H), jnp.float32),
        mesh=plsc.VectorSubcoreMesh(core_axis_name="c", subcore_axis_name="s",
                                    num_cores=2, num_subcores=NTILES),
        scratch_types=[
            tuple(pltpu.VMEM((SB, 2, B), jnp.int32) for _ in range(2)),
            tuple(pltpu.SemaphoreType.DMA for _ in range(2)),
            tuple(pltpu.VMEM((B, H), jnp.float32) for _ in range(NBUF)),
            tuple(pltpu.SemaphoreType.DMA for _ in range(NBUF)),
            pltpu.VMEM_SHARED((N, H), jnp.float32),
        ],
    )


def kernel(x, edge_index, edge_type, node_position, W_rel, W_self, b):
    src = edge_index[0]
    dst = edge_index[1]
    tidx = edge_type * N + src
    ge = jnp.stack([tidx, tidx + NROW]).reshape(2, NTILES, NSB, SB, B)
    de = jnp.broadcast_to(dst, (2, E)).reshape(2, NTILES, NSB, SB, B)
    idx = jnp.stack([ge, de], axis=-2)
    w_cat = jnp.concatenate([W_rel, W_self[:, None]], axis=1)

    outs = []
    yz = _tc_tables(x, w_cat[0], b[0])
    acc2 = _make_sc_scatter()(yz.reshape(2 * NROW, H), idx)
    for l in range(1, L):
        yz, h_prev = _tc_tables_fused(acc2, w_cat[l], b[l])
        outs.append(h_prev)
        acc2 = _make_sc_scatter()(yz.reshape(2 * NROW, H), idx)
    outs.append(_tc_relu_concat(acc2))
    node_feature = jnp.concatenate(outs, axis=-1)
    return node_feature, node_position


# speedup vs baseline: 1.2254x; 1.0390x over previous
import functools

import jax
import jax.numpy as jnp
from jax import lax
from jax.experimental import pallas as pl
from jax.experimental.pallas import tpu as pltpu
from jax.experimental.pallas import tpu_sc as plsc

N = 10000
E = 160000
D = 256
R = 7
L = 3
H = 128
NTILES = 16
ET = E // NTILES
B = 125
NB = ET // B
SB = 10
NSB = NB // SB
CHUNK = 640
LAST = N - CHUNK * (NTILES - 1)


NROW = (R + 1) * N


def _yz_body(h_ref, w_ref, b_ref, y_ref):
    acc = jnp.dot(h_ref[...].astype(jnp.bfloat16),
                  w_ref[0].astype(jnp.bfloat16),
                  preferred_element_type=jnp.float32)
    sel = jnp.where(pl.program_id(1) == R, 1.0, 0.0)
    acc = acc + sel * b_ref[...]
    y_ref[0] = acc[:, :H]
    y_ref[1] = acc[:, H:]


def _tc_tables(h, w_cat, b):
    bn = 2000
    nb = N // bn
    return pl.pallas_call(
        _yz_body,
        grid=(nb, R + 1),
        in_specs=[
            pl.BlockSpec((bn, D), lambda i, r: (i, 0)),
            pl.BlockSpec((1, D, D), lambda i, r: (r, 0, 0)),
            pl.BlockSpec((1, D), lambda i, r: (0, 0)),
        ],
        out_specs=pl.BlockSpec((2, bn, H), lambda i, r: (0, r * nb + i, 0)),
        out_shape=jax.ShapeDtypeStruct((2, NROW, H), jnp.float32),
    )(h, w_cat, b.reshape(1, D))


def _yzc_body(a_ref, w_ref, b_ref, *refs):
    y_ref, h_ref = refs[-2], refs[-1]
    h = jnp.maximum(jnp.concatenate([a_ref[0], a_ref[1]], axis=-1), 0.0)
    h_ref[...] = h
    acc = jnp.dot(h.astype(jnp.bfloat16), w_ref[0].astype(jnp.bfloat16),
                  preferred_element_type=jnp.float32)
    sel = jnp.where(pl.program_id(1) == R, 1.0, 0.0)
    acc = acc + sel * b_ref[...]
    y_ref[0] = acc[:, :H]
    y_ref[1] = acc[:, H:]


def _tc_tables_fused(acc2, w_cat, b, col, nf_prev):
    bn = 2000
    nb = N // bn
    args = [acc2, w_cat, b.reshape(1, D)]
    in_specs = [
        pl.BlockSpec((2, bn, H), lambda i, r: (0, i, 0)),
        pl.BlockSpec((1, D, D), lambda i, r: (r, 0, 0)),
        pl.BlockSpec((1, D), lambda i, r: (0, 0)),
    ]
    aliases = {}
    if nf_prev is not None:
        args.append(nf_prev)
        in_specs.append(pl.BlockSpec(memory_space=pltpu.MemorySpace.HBM))
        aliases = {3: 1}
    return pl.pallas_call(
        functools.partial(_yzc_body),
        grid=(nb, R + 1),
        in_specs=in_specs,
        out_specs=[
            pl.BlockSpec((2, bn, H), lambda i, r: (0, r * nb + i, 0)),
            pl.BlockSpec((bn, D), lambda i, r, col=col: (i, col)),
        ],
        out_shape=[
            jax.ShapeDtypeStruct((2, NROW, H), jnp.float32),
            jax.ShapeDtypeStruct((N, L * D), jnp.float32),
        ],
        input_output_aliases=aliases,
    )(*args)


def _combine_body(a_ref, nf_ref, o_ref):
    del nf_ref
    o_ref[...] = jnp.maximum(
        jnp.concatenate([a_ref[0], a_ref[1]], axis=-1), 0.0)


def _tc_relu_concat(acc2, nf_prev):
    bn = 2000
    nb = N // bn
    return pl.pallas_call(
        _combine_body,
        grid=(nb,),
        in_specs=[
            pl.BlockSpec((2, bn, H), lambda i: (0, i, 0)),
            pl.BlockSpec(memory_space=pltpu.MemorySpace.HBM),
        ],
        out_specs=pl.BlockSpec((bn, D), lambda i: (i, L - 1)),
        out_shape=jax.ShapeDtypeStruct((N, L * D), jnp.float32),
        input_output_aliases={1: 0},
    )(acc2, nf_prev)


NBUF = 2


def _sc_scatter_body(y_hbm, idx_hbm, out_hbm,
                     ibufs, isems, bufs, sems, acc):
    c = lax.axis_index("c")
    s = lax.axis_index("s")
    pltpu.async_copy(idx_hbm.at[c, s, 0], ibufs[0], isems[0])
    zbase = c * NROW + R * N

    @pl.when(s < NTILES - 1)
    def _():
        pltpu.sync_copy(y_hbm.at[pl.ds(zbase + s * CHUNK, CHUNK)],
                        acc.at[pl.ds(s * CHUNK, CHUNK)])

    @pl.when(s == NTILES - 1)
    def _():
        pltpu.sync_copy(
            y_hbm.at[pl.ds(zbase + (NTILES - 1) * CHUNK, LAST)],
            acc.at[pl.ds((NTILES - 1) * CHUNK, LAST)])

    pltpu.make_async_copy(idx_hbm.at[c, s, 0], ibufs[0], isems[0]).wait()
    pltpu.async_copy(idx_hbm.at[c, s, 1], ibufs[1], isems[1])
    plsc.subcore_barrier()

    for p in range(NBUF):
        pltpu.async_copy(y_hbm.at[ibufs[0].at[p, 0]], bufs[p], sems[p])

    for k in range(NSB):
        ib = ibufs[k % 2]
        ibn = ibufs[(k + 1) % 2]

        def inner(b2, carry, ib=ib):
            for p in range(NBUF):
                bl = b2 * NBUF + p
                pltpu.make_async_copy(y_hbm.at[ib.at[bl, 0]],
                                      bufs[p], sems[p]).wait()
                pltpu.sync_copy(bufs[p], acc.at[ib.at[bl, 1]], add=True)
                pltpu.async_copy(y_hbm.at[ib.at[bl + NBUF, 0]],
                                 bufs[p], sems[p])
            return carry

        lax.fori_loop(0, (SB - NBUF) // NBUF, inner, 0)
        if k + 1 < NSB:
            pltpu.make_async_copy(idx_hbm.at[c, s, k + 1], ibn,
                                  isems[(k + 1) % 2]).wait()
        for p in range(NBUF):
            bl = SB - NBUF + p
            pltpu.make_async_copy(y_hbm.at[ib.at[bl, 0]],
                                  bufs[p], sems[p]).wait()
            pltpu.sync_copy(bufs[p], acc.at[ib.at[bl, 1]], add=True)
            if k + 1 < NSB:
                pltpu.async_copy(y_hbm.at[ibn.at[p, 0]], bufs[p], sems[p])
        if k + 2 < NSB:
            pltpu.async_copy(idx_hbm.at[c, s, k + 2], ib, isems[k % 2])
    plsc.subcore_barrier()

    @pl.when(s < NTILES - 1)
    def _():
        pltpu.sync_copy(acc.at[pl.ds(s * CHUNK, CHUNK)],
                        out_hbm.at[c, pl.ds(s * CHUNK, CHUNK)])

    @pl.when(s == NTILES - 1)
    def _():
        pltpu.sync_copy(acc.at[pl.ds((NTILES - 1) * CHUNK, LAST)],
                        out_hbm.at[c, pl.ds((NTILES - 1) * CHUNK, LAST)])


@functools.cache
def _make_sc_scatter():
    return pl.kernel(
        _sc_scatter_body,
        out_type=jax.ShapeDtypeStruct((2, N, H), jnp.float32),
        mesh=plsc.VectorSubcoreMesh(core_axis_name="c", subcore_axis_name="s",
                                    num_cores=2, num_subcores=NTILES),
        scratch_types=[
            tuple(pltpu.VMEM((SB, 2, B), jnp.int32) for _ in range(2)),
            tuple(pltpu.SemaphoreType.DMA for _ in range(2)),
            tuple(pltpu.VMEM((B, H), jnp.float32) for _ in range(NBUF)),
            tuple(pltpu.SemaphoreType.DMA for _ in range(NBUF)),
            pltpu.VMEM_SHARED((N, H), jnp.float32),
        ],
    )


def kernel(x, edge_index, edge_type, node_position, W_rel, W_self, b):
    src = edge_index[0]
    dst = edge_index[1]
    tidx = edge_type * N + src
    ge = jnp.stack([tidx, tidx + NROW]).reshape(2, NTILES, NSB, SB, B)
    de = jnp.broadcast_to(dst, (2, E)).reshape(2, NTILES, NSB, SB, B)
    idx = jnp.stack([ge, de], axis=-2)
    w_cat = jnp.concatenate([W_rel, W_self[:, None]], axis=1)

    yz = _tc_tables(x, w_cat[0], b[0])
    acc2 = _make_sc_scatter()(yz.reshape(2 * NROW, H), idx)
    nf = None
    for l in range(1, L):
        yz, nf = _tc_tables_fused(acc2, w_cat[l], b[l], l - 1, nf)
        acc2 = _make_sc_scatter()(yz.reshape(2 * NROW, H), idx)
    node_feature = _tc_relu_concat(acc2, nf)
    return node_feature, node_position


# speedup vs baseline: 1.2263x; 1.0007x over previous
import functools

import jax
import jax.numpy as jnp
from jax import lax
from jax.experimental import pallas as pl
from jax.experimental.pallas import tpu as pltpu
from jax.experimental.pallas import tpu_sc as plsc

N = 10000
E = 160000
D = 256
R = 7
L = 3
H = 128
NTILES = 16
ET = E // NTILES
B = 125
NB = ET // B
SB = 10
NSB = NB // SB
CHUNK = 640
LAST = N - CHUNK * (NTILES - 1)


NROW = (R + 1) * N


def _yz_body(h_ref, w_ref, b_ref, y_ref):
    acc = jnp.dot(h_ref[...].astype(jnp.bfloat16),
                  w_ref[0].astype(jnp.bfloat16),
                  preferred_element_type=jnp.float32)
    sel = jnp.where(pl.program_id(1) == R, 1.0, 0.0)
    acc = acc + sel * b_ref[...]
    y_ref[0] = acc[:, :H]
    y_ref[1] = acc[:, H:]


def _tc_tables(h, w_cat, b):
    bn = 2000
    nb = N // bn
    return pl.pallas_call(
        _yz_body,
        grid=(nb, R + 1),
        in_specs=[
            pl.BlockSpec((bn, D), lambda i, r: (i, 0)),
            pl.BlockSpec((1, D, D), lambda i, r: (r, 0, 0)),
            pl.BlockSpec((1, D), lambda i, r: (0, 0)),
        ],
        out_specs=pl.BlockSpec((2, bn, H), lambda i, r: (0, r * nb + i, 0)),
        out_shape=jax.ShapeDtypeStruct((2, NROW, H), jnp.float32),
    )(h, w_cat, b.reshape(1, D))


def _yzc_body(a_ref, w_ref, b_ref, *refs):
    y_ref, h_ref = refs[-2], refs[-1]
    h = jnp.maximum(jnp.concatenate([a_ref[0], a_ref[1]], axis=-1), 0.0)
    h_ref[...] = h
    acc = jnp.dot(h.astype(jnp.bfloat16), w_ref[0].astype(jnp.bfloat16),
                  preferred_element_type=jnp.float32)
    sel = jnp.where(pl.program_id(1) == R, 1.0, 0.0)
    acc = acc + sel * b_ref[...]
    y_ref[0] = acc[:, :H]
    y_ref[1] = acc[:, H:]


def _tc_tables_fused(acc2, w_cat, b, col, nf_prev):
    bn = 2000
    nb = N // bn
    args = [acc2, w_cat, b.reshape(1, D)]
    in_specs = [
        pl.BlockSpec((2, bn, H), lambda i, r: (0, i, 0)),
        pl.BlockSpec((1, D, D), lambda i, r: (r, 0, 0)),
        pl.BlockSpec((1, D), lambda i, r: (0, 0)),
    ]
    aliases = {}
    if nf_prev is not None:
        args.append(nf_prev)
        in_specs.append(pl.BlockSpec(memory_space=pltpu.MemorySpace.HBM))
        aliases = {3: 1}
    return pl.pallas_call(
        _yzc_body,
        grid=(nb, R + 1),
        in_specs=in_specs,
        out_specs=[
            pl.BlockSpec((2, bn, H), lambda i, r: (0, r * nb + i, 0)),
            pl.BlockSpec((bn, D), lambda i, r, col=col: (i, col)),
        ],
        out_shape=[
            jax.ShapeDtypeStruct((2, NROW, H), jnp.float32),
            jax.ShapeDtypeStruct((N, L * D), jnp.float32),
        ],
        input_output_aliases=aliases,
    )(*args)


def _combine_body(a_ref, nf_ref, o_ref):
    del nf_ref
    o_ref[...] = jnp.maximum(
        jnp.concatenate([a_ref[0], a_ref[1]], axis=-1), 0.0)


def _tc_relu_concat(acc2, nf_prev):
    bn = 2000
    nb = N // bn
    return pl.pallas_call(
        _combine_body,
        grid=(nb,),
        in_specs=[
            pl.BlockSpec((2, bn, H), lambda i: (0, i, 0)),
            pl.BlockSpec(memory_space=pltpu.MemorySpace.HBM),
        ],
        out_specs=pl.BlockSpec((bn, D), lambda i: (i, L - 1)),
        out_shape=jax.ShapeDtypeStruct((N, L * D), jnp.float32),
        input_output_aliases={1: 0},
    )(acc2, nf_prev)


NBUF = 2


def _sc_scatter_body(y_hbm, idx_hbm, out_hbm,
                     ibufs, isems, bufs, sems, acc):
    c = lax.axis_index("c")
    s = lax.axis_index("s")
    pltpu.async_copy(idx_hbm.at[c, s, 0], ibufs[0], isems[0])
    zbase = c * NROW + R * N

    @pl.when(s < NTILES - 1)
    def _():
        pltpu.sync_copy(y_hbm.at[pl.ds(zbase + s * CHUNK, CHUNK)],
                        acc.at[pl.ds(s * CHUNK, CHUNK)])

    @pl.when(s == NTILES - 1)
    def _():
        pltpu.sync_copy(
            y_hbm.at[pl.ds(zbase + (NTILES - 1) * CHUNK, LAST)],
            acc.at[pl.ds((NTILES - 1) * CHUNK, LAST)])

    pltpu.make_async_copy(idx_hbm.at[c, s, 0], ibufs[0], isems[0]).wait()
    pltpu.async_copy(idx_hbm.at[c, s, 1], ibufs[1], isems[1])
    plsc.subcore_barrier()

    for p in range(NBUF):
        pltpu.async_copy(y_hbm.at[ibufs[0].at[p, 0]], bufs[p], sems[p])

    for k in range(NSB):
        ib = ibufs[k % 2]
        ibn = ibufs[(k + 1) % 2]

        def inner(b2, carry, ib=ib):
            for p in range(NBUF):
                bl = b2 * NBUF + p
                pltpu.make_async_copy(y_hbm.at[ib.at[bl, 0]],
                                      bufs[p], sems[p]).wait()
                pltpu.sync_copy(bufs[p], acc.at[ib.at[bl, 1]], add=True)
                pltpu.async_copy(y_hbm.at[ib.at[bl + NBUF, 0]],
                                 bufs[p], sems[p])
            return carry

        lax.fori_loop(0, (SB - NBUF) // NBUF, inner, 0)
        if k + 1 < NSB:
            pltpu.make_async_copy(idx_hbm.at[c, s, k + 1], ibn,
                                  isems[(k + 1) % 2]).wait()
        for p in range(NBUF):
            bl = SB - NBUF + p
            pltpu.make_async_copy(y_hbm.at[ib.at[bl, 0]],
                                  bufs[p], sems[p]).wait()
            pltpu.sync_copy(bufs[p], acc.at[ib.at[bl, 1]], add=True)
            if k + 1 < NSB:
                pltpu.async_copy(y_hbm.at[ibn.at[p, 0]], bufs[p], sems[p])
        if k + 2 < NSB:
            pltpu.async_copy(idx_hbm.at[c, s, k + 2], ib, isems[k % 2])
    plsc.subcore_barrier()

    @pl.when(s < NTILES - 1)
    def _():
        pltpu.sync_copy(acc.at[pl.ds(s * CHUNK, CHUNK)],
                        out_hbm.at[c, pl.ds(s * CHUNK, CHUNK)])

    @pl.when(s == NTILES - 1)
    def _():
        pltpu.sync_copy(acc.at[pl.ds((NTILES - 1) * CHUNK, LAST)],
                        out_hbm.at[c, pl.ds((NTILES - 1) * CHUNK, LAST)])


@functools.cache
def _make_sc_scatter():
    return pl.kernel(
        _sc_scatter_body,
        out_type=jax.ShapeDtypeStruct((2, N, H), jnp.float32),
        mesh=plsc.VectorSubcoreMesh(core_axis_name="c", subcore_axis_name="s",
                                    num_cores=2, num_subcores=NTILES),
        scratch_types=[
            tuple(pltpu.VMEM((SB, 2, B), jnp.int32) for _ in range(2)),
            tuple(pltpu.SemaphoreType.DMA for _ in range(2)),
            tuple(pltpu.VMEM((B, H), jnp.float32) for _ in range(NBUF)),
            tuple(pltpu.SemaphoreType.DMA for _ in range(NBUF)),
            pltpu.VMEM_SHARED((N, H), jnp.float32),
        ],
    )


def kernel(x, edge_index, edge_type, node_position, W_rel, W_self, b):
    src = edge_index[0]
    dst = edge_index[1]
    tidx = edge_type * N + src
    ge = jnp.stack([tidx, tidx + NROW]).reshape(2, NTILES, NSB, SB, B)
    de = jnp.broadcast_to(dst, (2, E)).reshape(2, NTILES, NSB, SB, B)
    idx = jnp.stack([ge, de], axis=-2)
    w_cat = jnp.concatenate([W_rel, W_self[:, None]], axis=1)

    yz = _tc_tables(x, w_cat[0], b[0])
    acc2 = _make_sc_scatter()(yz.reshape(2 * NROW, H), idx)
    nf = None
    for l in range(1, L):
        yz, nf = _tc_tables_fused(acc2, w_cat[l], b[l], l - 1, nf)
        acc2 = _make_sc_scatter()(yz.reshape(2 * NROW, H), idx)
    node_feature = _tc_relu_concat(acc2, nf)
    return node_feature, node_position
